# baseline (speedup 1.0000x reference)
import functools

import jax
import jax.numpy as jnp
from jax import lax
from jax.experimental import pallas as pl
from jax.experimental.pallas import tpu as pltpu
from jax.experimental.pallas import tpu_sc as plsc

N = 10000
E = 320000
D = 128
HD = D // 2
R = 16

NC = 2
NS = 16
CHUNK = 128
NCHUNKS = E // CHUNK
CHUNKS_MAIN = NCHUNKS // NS
NLEFT = NCHUNKS - NS * CHUNKS_MAIN
ACH = CHUNK * R // 128
N_PAD = 10240
ROWS_PER_TILE = N_PAD // NS

BCHUNKS = NCHUNKS // NC
BMAIN = BCHUNKS // NS
BLEFT = BCHUNKS - NS * BMAIN


def _zero16():
    return jnp.zeros((16,), jnp.float32)


def _sc_a_body(ha_hbm, hb_hbm, src_hbm, dst_hbm,
               out_h, out_d,
               srcall, dstall, rows0, rows1, rows2, rows3, ones,
               semg0, semg1, semg2, semg3, sems0, sems1, sems2, sems3,
               hacc, dacc):
    c = lax.axis_index("c")
    s = lax.axis_index("s")
    rows = (rows0, rows1, rows2, rows3)
    semg = (semg0, semg1, semg2, semg3)
    sems = (sems0, sems1, sems2, sems3)

    zero16 = jnp.zeros((16,), jnp.float32)
    one16 = jnp.ones((16,), jnp.float32)

    def fill_zero(i, _):
        for j in range(HD // 16):
            rows0[i, pl.ds(j * 16, 16)] = zero16
        ones[i, :] = zero16
        return 0

    lax.fori_loop(0, CHUNK, fill_zero, 0)

    base = s * ROWS_PER_TILE
    for k in range(ROWS_PER_TILE // CHUNK):
        off = k * CHUNK
        pltpu.sync_copy(rows0, hacc.at[pl.ds(base + off, CHUNK)])
        pltpu.sync_copy(ones, dacc.at[pl.ds(base + off, CHUNK)])

    def fill_ones(i, _):
        ones[i, :] = one16
        return 0

    lax.fori_loop(0, CHUNK, fill_ones, 0)

    chunk0 = s * CHUNKS_MAIN
    pltpu.sync_copy(src_hbm.at[pl.ds(chunk0, CHUNKS_MAIN)], srcall)
    pltpu.sync_copy(dst_hbm.at[pl.ds(chunk0, CHUNKS_MAIN)], dstall)
    plsc.subcore_barrier()

    def run_pipeline(h_half, use_deg):
        def fire_gather(g, b):
            pltpu.async_copy(h_half.at[srcall.at[g]], rows[b], semg[b])

        def fire_scatter(g, b):
            pltpu.make_async_copy(h_half.at[srcall.at[0]], rows[b],
                                  semg[b]).wait()
            pltpu.async_copy(rows[b], hacc.at[dstall.at[g]], sems[b],
                             add=True)
            if use_deg:
                pltpu.async_copy(ones, dacc.at[dstall.at[g]], sems[b],
                                 add=True)

        def wait_scatter(b):
            pltpu.make_async_copy(rows[b], hacc.at[dstall.at[0]],
                                  sems[b]).wait()
            if use_deg:
                pltpu.make_async_copy(ones, dacc.at[dstall.at[0]],
                                      sems[b]).wait()

        fire_gather(0, 0)
        fire_gather(1, 1)
        fire_scatter(0, 0)
        fire_gather(2, 2)
        fire_scatter(1, 1)
        fire_gather(3, 3)

        @pl.loop(4, CHUNKS_MAIN, step=4)
        def _(gg):
            for k in range(4):
                g = gg + k
                wait_scatter(k)
                fire_scatter(g - 2, (k + 2) % 4)
                fire_gather(g, k)

        fire_scatter(CHUNKS_MAIN - 2, (CHUNKS_MAIN - 2) % 4)
        fire_scatter(CHUNKS_MAIN - 1, (CHUNKS_MAIN - 1) % 4)
        for b in range(4):
            wait_scatter(b)

        @pl.when(s < NLEFT)
        def _():
            xchunk = NS * CHUNKS_MAIN + s
            pltpu.sync_copy(src_hbm.at[pl.ds(xchunk, 1)],
                            srcall.at[pl.ds(0, 1)])
            pltpu.sync_copy(dst_hbm.at[pl.ds(xchunk, 1)],
                            dstall.at[pl.ds(0, 1)])
            pltpu.sync_copy(h_half.at[srcall.at[0]], rows0)
            pltpu.sync_copy(rows0, hacc.at[dstall.at[0]], add=True)
            if use_deg:
                pltpu.sync_copy(ones, dacc.at[dstall.at[0]], add=True)

    @pl.when(c == 0)
    def _():
        run_pipeline(ha_hbm, False)

    @pl.when(c == 1)
    def _():
        run_pipeline(hb_hbm, True)

    plsc.subcore_barrier()

    hs_src = hacc.at[pl.ds(base, ROWS_PER_TILE)]

    @pl.when(c == 0)
    def _():
        pltpu.sync_copy(hs_src, out_h.at[0, pl.ds(base, ROWS_PER_TILE)])

    @pl.when(c == 1)
    def _():
        pltpu.sync_copy(hs_src, out_h.at[1, pl.ds(base, ROWS_PER_TILE)])
        pltpu.sync_copy(dacc.at[pl.ds(base, ROWS_PER_TILE)],
                        out_d.at[pl.ds(base, ROWS_PER_TILE)])


def _sc_b_body(attr_hbm, dst_hbm,
               out_a,
               dstall, attrv0, attrv1, attrv2, attrs0, attrs1, attrs2,
               sema0, sema1, sema2, sems0, sems1, sems2,
               sacc):
    c = lax.axis_index("c")
    s = lax.axis_index("s")
    attrv = (attrv0, attrv1, attrv2)
    attrs = (attrs0, attrs1, attrs2)
    sema = (sema0, sema1, sema2)
    sems = (sems0, sems1, sems2)

    zero16 = jnp.zeros((16,), jnp.float32)

    def fill_zero(i, _):
        attrs0[i, :] = zero16
        return 0

    lax.fori_loop(0, CHUNK, fill_zero, 0)

    base = s * ROWS_PER_TILE
    for k in range(ROWS_PER_TILE // CHUNK):
        pltpu.sync_copy(attrs0, sacc.at[pl.ds(base + k * CHUNK, CHUNK)])

    chunk0 = c * BCHUNKS + s * BMAIN
    pltpu.sync_copy(dst_hbm.at[pl.ds(chunk0, BMAIN)], dstall)
    plsc.subcore_barrier()

    def fire_stage(g, b):
        gc = jnp.minimum(g, BMAIN - 1)
        pltpu.async_copy(
            attr_hbm.at[pl.ds((chunk0 + gc) * ACH, ACH)], attrv[b],
            sema[b])

    def fire_scatter(g, b):
        pltpu.make_async_copy(attr_hbm.at[pl.ds(0, ACH)], attrv[b],
                              sema[b]).wait()
        for r in range(ACH):
            for j in range(128 // R):
                attrs[b][(128 // R) * r + j, :] = \
                    attrv[b][r, pl.ds(R * j, R)]
        pltpu.async_copy(attrs[b], sacc.at[dstall.at[g]], sems[b],
                         add=True)

    def wait_scatter(b):
        pltpu.make_async_copy(attrs[b], sacc.at[dstall.at[0]],
                              sems[b]).wait()

    fire_stage(0, 0)
    fire_stage(1, 1)
    fire_scatter(0, 0)
    fire_stage(2, 2)

    @pl.loop(3, BMAIN, step=3)
    def _(gg):
        for k in range(3):
            g = gg + k
            wait_scatter(k)
            fire_scatter(g - 2, (k + 1) % 3)
            fire_stage(g, k)

    fire_scatter(BMAIN - 2, (BMAIN - 2) % 3)
    fire_scatter(BMAIN - 1, (BMAIN - 1) % 3)
    for b in range(3):
        wait_scatter(b)

    @pl.when(s < BLEFT)
    def _():
        xchunk = c * BCHUNKS + NS * BMAIN + s
        pltpu.sync_copy(dst_hbm.at[pl.ds(xchunk, 1)],
                        dstall.at[pl.ds(0, 1)])
        pltpu.sync_copy(attr_hbm.at[pl.ds(xchunk * ACH, ACH)], attrv0)
        for r in range(ACH):
            for j in range(128 // R):
                attrs1[(128 // R) * r + j, :] = attrv0[r, pl.ds(R * j, R)]
        pltpu.sync_copy(attrs1, sacc.at[dstall.at[0]], add=True)

    plsc.subcore_barrier()
    pltpu.sync_copy(sacc.at[pl.ds(base, ROWS_PER_TILE)],
                    out_a.at[c, pl.ds(base, ROWS_PER_TILE)])


@functools.cache
def _make_sc_a():
  return functools.partial(
    pl.kernel,
    out_type=(
        jax.ShapeDtypeStruct((NC, N_PAD, HD), jnp.float32),
        jax.ShapeDtypeStruct((N_PAD, 16), jnp.float32),
    ),
    mesh=plsc.VectorSubcoreMesh(core_axis_name="c", subcore_axis_name="s"),
    compiler_params=pltpu.CompilerParams(use_tc_tiling_on_sc=False),
    scratch_types=(
        pltpu.VMEM((CHUNKS_MAIN, CHUNK), jnp.int32),
        pltpu.VMEM((CHUNKS_MAIN, CHUNK), jnp.int32),
        pltpu.VMEM((CHUNK, HD), jnp.float32),
        pltpu.VMEM((CHUNK, HD), jnp.float32),
        pltpu.VMEM((CHUNK, HD), jnp.float32),
        pltpu.VMEM((CHUNK, HD), jnp.float32),
        pltpu.VMEM((CHUNK, 16), jnp.float32),
        pltpu.SemaphoreType.DMA,
        pltpu.SemaphoreType.DMA,
        pltpu.SemaphoreType.DMA,
        pltpu.SemaphoreType.DMA,
        pltpu.SemaphoreType.DMA,
        pltpu.SemaphoreType.DMA,
        pltpu.SemaphoreType.DMA,
        pltpu.SemaphoreType.DMA,
        pltpu.VMEM_SHARED((N_PAD, HD), jnp.float32),
        pltpu.VMEM_SHARED((N_PAD, 16), jnp.float32),
    ),
  )(_sc_a_body)


@functools.cache
def _make_sc_b():
  return functools.partial(
    pl.kernel,
    out_type=jax.ShapeDtypeStruct((NC, N_PAD, R), jnp.float32),
    mesh=plsc.VectorSubcoreMesh(core_axis_name="c", subcore_axis_name="s"),
    compiler_params=pltpu.CompilerParams(use_tc_tiling_on_sc=False),
    scratch_types=(
        pltpu.VMEM((BMAIN, CHUNK), jnp.int32),
        pltpu.VMEM((ACH, 128), jnp.float32),
        pltpu.VMEM((ACH, 128), jnp.float32),
        pltpu.VMEM((ACH, 128), jnp.float32),
        pltpu.VMEM((CHUNK, R), jnp.float32),
        pltpu.VMEM((CHUNK, R), jnp.float32),
        pltpu.VMEM((CHUNK, R), jnp.float32),
        pltpu.SemaphoreType.DMA,
        pltpu.SemaphoreType.DMA,
        pltpu.SemaphoreType.DMA,
        pltpu.SemaphoreType.DMA,
        pltpu.SemaphoreType.DMA,
        pltpu.SemaphoreType.DMA,
        pltpu.VMEM_SHARED((N_PAD, R), jnp.float32),
    ),
  )(_sc_b_body)


def _tc_body(h, q, hs2, ab2, dg_,
             W_query, b_query, W_mem, b_mem, Wm1, Wm2, b_msg,
             Wa1, Wa2, Wa3, b_all, out):
    hp = lax.Precision.HIGHEST
    A = lax.dot(W_query[...], Wa1[...], precision=hp)
    B = lax.dot(W_mem[...], Wa2[...], precision=hp)
    C1 = lax.dot(Wm1[...], Wa3[...], precision=hp)
    C2 = lax.dot(Wm2[...], Wa3[...], precision=hp)
    cb = (lax.dot(b_query[...], Wa1[...], precision=hp)
          + lax.dot(b_mem[...], Wa2[...], precision=hp)
          + b_all[...])
    mb = lax.dot(b_msg[...], Wa3[...], precision=hp)
    dg = dg_[:, 0:1]
    hs = jnp.concatenate([hs2[0], hs2[1]], axis=-1)
    as_ = ab2[0] + ab2[1]
    pre = (lax.dot(q[...], A) + lax.dot(h[...], B)
           + lax.dot(hs, C1) + lax.dot(as_, C2)
           + dg * mb + cb)
    new_h = jnp.maximum(pre, 0.0)
    out[...] = jnp.where(dg > 0.0, new_h, h[...])


def kernel(h, q, edge_index, edge_attr, W_msg, b_msg, W_mem, b_mem,
           W_query, b_query, W_all, b_all):
    edge_index = edge_index.astype(jnp.int32)
    src2d = edge_index[0].reshape(E // CHUNK, CHUNK)
    dst2d = edge_index[1].reshape(E // CHUNK, CHUNK)
    ha = h[:, :HD]
    hb = h[:, HD:]
    attr_packed = edge_attr.reshape(E * R // 128, 128)

    hs2, dg_ = _make_sc_a()(ha, hb, src2d, dst2d)
    ab2 = _make_sc_b()(attr_packed, dst2d)

    Wm1 = W_msg[:D]
    Wm2 = W_msg[D:]
    Wa1 = W_all[:D]
    Wa2 = W_all[D:2 * D]
    Wa3 = W_all[2 * D:]

    BN = 1000
    grid = N // BN
    row_block = lambda r, c: pl.BlockSpec((r, c), lambda i: (i, 0))
    full = lambda *shape: pl.BlockSpec(shape, lambda i: (0,) * len(shape))

    out = pl.pallas_call(
        _tc_body,
        grid=(grid,),
        in_specs=[
            row_block(BN, D),
            row_block(BN, D),
            pl.BlockSpec((NC, BN, HD), lambda i: (0, i, 0)),
            pl.BlockSpec((NC, BN, R), lambda i: (0, i, 0)),
            row_block(BN, 16),
            full(D, D),
            full(1, D),
            full(D, D),
            full(1, D),
            full(D, D),
            full(R, D),
            full(1, D),
            full(D, D),
            full(D, D),
            full(D, D),
            full(1, D),
        ],
        out_specs=row_block(BN, D),
        out_shape=jax.ShapeDtypeStruct((N, D), jnp.float32),
    )(h, q, hs2, ab2, dg_,
      W_query, b_query.reshape(1, D), W_mem, b_mem.reshape(1, D),
      Wm1, Wm2, b_msg.reshape(1, D),
      Wa1, Wa2, Wa3, b_all.reshape(1, D))
    return out


# speedup vs baseline: 1.0191x; 1.0191x over previous
import functools

import jax
import jax.numpy as jnp
from jax import lax
from jax.experimental import pallas as pl
from jax.experimental.pallas import tpu as pltpu
from jax.experimental.pallas import tpu_sc as plsc

N = 10000
E = 320000
D = 128
HD = D // 2
R = 16

NC = 2
NS = 16
CHUNK = 128
NCHUNKS = E // CHUNK
CHUNKS_MAIN = NCHUNKS // NS
NLEFT = NCHUNKS - NS * CHUNKS_MAIN
ACH = CHUNK * R // 128
N_PAD = 10240
ROWS_PER_TILE = N_PAD // NS

BCHUNKS = NCHUNKS // NC
BMAIN = BCHUNKS // NS
BLEFT = BCHUNKS - NS * BMAIN


def _zero16():
    return jnp.zeros((16,), jnp.float32)


def _sc_a_body(ha_hbm, hb_hbm, src_hbm, dst_hbm,
               out_h, out_d,
               srcall, dstall, rows0, rows1, rows2, ones,
               semg0, semg1, semg2, sems0, sems1, sems2,
               hacc, dacc):
    c = lax.axis_index("c")
    s = lax.axis_index("s")
    rows = (rows0, rows1, rows2)
    semg = (semg0, semg1, semg2)
    sems = (sems0, sems1, sems2)

    zero16 = jnp.zeros((16,), jnp.float32)
    one16 = jnp.ones((16,), jnp.float32)

    def fill_zero(i, _):
        for j in range(HD // 16):
            rows0[i, pl.ds(j * 16, 16)] = zero16
        ones[i, :] = zero16
        return 0

    lax.fori_loop(0, CHUNK, fill_zero, 0)

    base = s * ROWS_PER_TILE
    for k in range(ROWS_PER_TILE // CHUNK):
        off = k * CHUNK
        pltpu.sync_copy(rows0, hacc.at[pl.ds(base + off, CHUNK)])
        pltpu.sync_copy(ones, dacc.at[pl.ds(base + off, CHUNK)])

    def fill_ones(i, _):
        ones[i, :] = one16
        return 0

    lax.fori_loop(0, CHUNK, fill_ones, 0)

    chunk0 = s * CHUNKS_MAIN
    pltpu.sync_copy(src_hbm.at[pl.ds(chunk0, CHUNKS_MAIN)], srcall)
    pltpu.sync_copy(dst_hbm.at[pl.ds(chunk0, CHUNKS_MAIN)], dstall)
    plsc.subcore_barrier()

    def run_pipeline(h_half, use_deg):
        def fire_gather(g, b):
            pltpu.async_copy(h_half.at[srcall.at[g]], rows[b], semg[b])

        def fire_scatter(g, b):
            pltpu.make_async_copy(h_half.at[srcall.at[0]], rows[b],
                                  semg[b]).wait()
            pltpu.async_copy(rows[b], hacc.at[dstall.at[g]], sems[b],
                             add=True)
            if use_deg:
                pltpu.async_copy(ones, dacc.at[dstall.at[g]], sems[b],
                                 add=True)

        def wait_scatter(b):
            pltpu.make_async_copy(rows[b], hacc.at[dstall.at[0]],
                                  sems[b]).wait()
            if use_deg:
                pltpu.make_async_copy(ones, dacc.at[dstall.at[0]],
                                      sems[b]).wait()

        fire_gather(0, 0)
        fire_gather(1, 1)
        fire_scatter(0, 0)
        fire_gather(2, 2)

        @pl.loop(3, CHUNKS_MAIN, step=3)
        def _(gg):
            for k in range(3):
                g = gg + k
                wait_scatter(k)
                fire_scatter(g - 2, (k + 1) % 3)
                fire_gather(g, k)

        fire_scatter(CHUNKS_MAIN - 2, (CHUNKS_MAIN - 2) % 3)
        fire_scatter(CHUNKS_MAIN - 1, (CHUNKS_MAIN - 1) % 3)
        for b in range(3):
            wait_scatter(b)

        @pl.when(s < NLEFT)
        def _():
            xchunk = NS * CHUNKS_MAIN + s
            pltpu.sync_copy(src_hbm.at[pl.ds(xchunk, 1)],
                            srcall.at[pl.ds(0, 1)])
            pltpu.sync_copy(dst_hbm.at[pl.ds(xchunk, 1)],
                            dstall.at[pl.ds(0, 1)])
            pltpu.sync_copy(h_half.at[srcall.at[0]], rows0)
            pltpu.sync_copy(rows0, hacc.at[dstall.at[0]], add=True)
            if use_deg:
                pltpu.sync_copy(ones, dacc.at[dstall.at[0]], add=True)

    @pl.when(c == 0)
    def _():
        run_pipeline(ha_hbm, False)

    @pl.when(c == 1)
    def _():
        run_pipeline(hb_hbm, True)

    plsc.subcore_barrier()

    hs_src = hacc.at[pl.ds(base, ROWS_PER_TILE)]

    @pl.when(c == 0)
    def _():
        pltpu.sync_copy(hs_src, out_h.at[0, pl.ds(base, ROWS_PER_TILE)])

    @pl.when(c == 1)
    def _():
        pltpu.sync_copy(hs_src, out_h.at[1, pl.ds(base, ROWS_PER_TILE)])
        pltpu.sync_copy(dacc.at[pl.ds(base, ROWS_PER_TILE)],
                        out_d.at[pl.ds(base, ROWS_PER_TILE)])


def _sc_b_body(attr_hbm, dst_hbm,
               out_a,
               dstall, attrv0, attrv1, attrv2, attrs0, attrs1, attrs2,
               sema0, sema1, sema2, sems0, sems1, sems2,
               sacc):
    c = lax.axis_index("c")
    s = lax.axis_index("s")
    attrv = (attrv0, attrv1, attrv2)
    attrs = (attrs0, attrs1, attrs2)
    sema = (sema0, sema1, sema2)
    sems = (sems0, sems1, sems2)

    zero16 = jnp.zeros((16,), jnp.float32)

    def fill_zero(i, _):
        attrs0[i, :] = zero16
        return 0

    lax.fori_loop(0, CHUNK, fill_zero, 0)

    base = s * ROWS_PER_TILE
    for k in range(ROWS_PER_TILE // CHUNK):
        pltpu.sync_copy(attrs0, sacc.at[pl.ds(base + k * CHUNK, CHUNK)])

    chunk0 = c * BCHUNKS + s * BMAIN
    pltpu.sync_copy(dst_hbm.at[pl.ds(chunk0, BMAIN)], dstall)
    plsc.subcore_barrier()

    def fire_stage(g, b):
        gc = jnp.minimum(g, BMAIN - 1)
        pltpu.async_copy(
            attr_hbm.at[pl.ds((chunk0 + gc) * ACH, ACH)], attrv[b],
            sema[b])

    def fire_scatter(g, b):
        pltpu.make_async_copy(attr_hbm.at[pl.ds(0, ACH)], attrv[b],
                              sema[b]).wait()
        for r in range(ACH):
            for j in range(128 // R):
                attrs[b][(128 // R) * r + j, :] = \
                    attrv[b][r, pl.ds(R * j, R)]
        pltpu.async_copy(attrs[b], sacc.at[dstall.at[g]], sems[b],
                         add=True)

    def wait_scatter(b):
        pltpu.make_async_copy(attrs[b], sacc.at[dstall.at[0]],
                              sems[b]).wait()

    fire_stage(0, 0)
    fire_stage(1, 1)
    fire_scatter(0, 0)
    fire_stage(2, 2)

    @pl.loop(3, BMAIN, step=3)
    def _(gg):
        for k in range(3):
            g = gg + k
            wait_scatter(k)
            fire_scatter(g - 2, (k + 1) % 3)
            fire_stage(g, k)

    fire_scatter(BMAIN - 2, (BMAIN - 2) % 3)
    fire_scatter(BMAIN - 1, (BMAIN - 1) % 3)
    for b in range(3):
        wait_scatter(b)

    @pl.when(s < BLEFT)
    def _():
        xchunk = c * BCHUNKS + NS * BMAIN + s
        pltpu.sync_copy(dst_hbm.at[pl.ds(xchunk, 1)],
                        dstall.at[pl.ds(0, 1)])
        pltpu.sync_copy(attr_hbm.at[pl.ds(xchunk * ACH, ACH)], attrv0)
        for r in range(ACH):
            for j in range(128 // R):
                attrs1[(128 // R) * r + j, :] = attrv0[r, pl.ds(R * j, R)]
        pltpu.sync_copy(attrs1, sacc.at[dstall.at[0]], add=True)

    plsc.subcore_barrier()
    pltpu.sync_copy(sacc.at[pl.ds(base, ROWS_PER_TILE)],
                    out_a.at[c, pl.ds(base, ROWS_PER_TILE)])


@functools.cache
def _make_sc_a():
  return functools.partial(
    pl.kernel,
    out_type=(
        jax.ShapeDtypeStruct((NC, N_PAD, HD), jnp.float32),
        jax.ShapeDtypeStruct((N_PAD, 16), jnp.float32),
    ),
    mesh=plsc.VectorSubcoreMesh(core_axis_name="c", subcore_axis_name="s"),
    compiler_params=pltpu.CompilerParams(use_tc_tiling_on_sc=False),
    scratch_types=(
        pltpu.VMEM((CHUNKS_MAIN, CHUNK), jnp.int32),
        pltpu.VMEM((CHUNKS_MAIN, CHUNK), jnp.int32),
        pltpu.VMEM((CHUNK, HD), jnp.float32),
        pltpu.VMEM((CHUNK, HD), jnp.float32),
        pltpu.VMEM((CHUNK, HD), jnp.float32),
        pltpu.VMEM((CHUNK, 16), jnp.float32),
        pltpu.SemaphoreType.DMA,
        pltpu.SemaphoreType.DMA,
        pltpu.SemaphoreType.DMA,
        pltpu.SemaphoreType.DMA,
        pltpu.SemaphoreType.DMA,
        pltpu.SemaphoreType.DMA,
        pltpu.VMEM_SHARED((N_PAD, HD), jnp.float32),
        pltpu.VMEM_SHARED((N_PAD, 16), jnp.float32),
    ),
  )(_sc_a_body)


@functools.cache
def _make_sc_b():
  return functools.partial(
    pl.kernel,
    out_type=jax.ShapeDtypeStruct((NC, N_PAD, R), jnp.float32),
    mesh=plsc.VectorSubcoreMesh(core_axis_name="c", subcore_axis_name="s"),
    compiler_params=pltpu.CompilerParams(use_tc_tiling_on_sc=False),
    scratch_types=(
        pltpu.VMEM((BMAIN, CHUNK), jnp.int32),
        pltpu.VMEM((ACH, 128), jnp.float32),
        pltpu.VMEM((ACH, 128), jnp.float32),
        pltpu.VMEM((ACH, 128), jnp.float32),
        pltpu.VMEM((CHUNK, R), jnp.float32),
        pltpu.VMEM((CHUNK, R), jnp.float32),
        pltpu.VMEM((CHUNK, R), jnp.float32),
        pltpu.SemaphoreType.DMA,
        pltpu.SemaphoreType.DMA,
        pltpu.SemaphoreType.DMA,
        pltpu.SemaphoreType.DMA,
        pltpu.SemaphoreType.DMA,
        pltpu.SemaphoreType.DMA,
        pltpu.VMEM_SHARED((N_PAD, R), jnp.float32),
    ),
  )(_sc_b_body)


def _tc_body(h, q, hs2, ab2, dg_,
             W_query, b_query, W_mem, b_mem, Wm1, Wm2, b_msg,
             Wa1, Wa2, Wa3, b_all, out):
    hp = lax.Precision.HIGHEST
    A = lax.dot(W_query[...], Wa1[...], precision=hp)
    B = lax.dot(W_mem[...], Wa2[...], precision=hp)
    C1 = lax.dot(Wm1[...], Wa3[...], precision=hp)
    C2 = lax.dot(Wm2[...], Wa3[...], precision=hp)
    cb = (lax.dot(b_query[...], Wa1[...], precision=hp)
          + lax.dot(b_mem[...], Wa2[...], precision=hp)
          + b_all[...])
    mb = lax.dot(b_msg[...], Wa3[...], precision=hp)
    dg = dg_[:, 0:1]
    hs = jnp.concatenate([hs2[0], hs2[1]], axis=-1)
    as_ = ab2[0] + ab2[1]
    pre = (lax.dot(q[...], A) + lax.dot(h[...], B)
           + lax.dot(hs, C1) + lax.dot(as_, C2)
           + dg * mb + cb)
    new_h = jnp.maximum(pre, 0.0)
    out[...] = jnp.where(dg > 0.0, new_h, h[...])


def kernel(h, q, edge_index, edge_attr, W_msg, b_msg, W_mem, b_mem,
           W_query, b_query, W_all, b_all):
    edge_index = edge_index.astype(jnp.int32)
    src2d = edge_index[0].reshape(E // CHUNK, CHUNK)
    dst2d = edge_index[1].reshape(E // CHUNK, CHUNK)
    ha = h[:, :HD]
    hb = h[:, HD:]
    attr_packed = edge_attr.reshape(E * R // 128, 128)

    hs2, dg_ = _make_sc_a()(ha, hb, src2d, dst2d)
    ab2 = _make_sc_b()(attr_packed, dst2d)

    Wm1 = W_msg[:D]
    Wm2 = W_msg[D:]
    Wa1 = W_all[:D]
    Wa2 = W_all[D:2 * D]
    Wa3 = W_all[2 * D:]

    BN = 1000
    grid = N // BN
    row_block = lambda r, c: pl.BlockSpec((r, c), lambda i: (i, 0))
    full = lambda *shape: pl.BlockSpec(shape, lambda i: (0,) * len(shape))

    out = pl.pallas_call(
        _tc_body,
        grid=(grid,),
        in_specs=[
            row_block(BN, D),
            row_block(BN, D),
            pl.BlockSpec((NC, BN, HD), lambda i: (0, i, 0)),
            pl.BlockSpec((NC, BN, R), lambda i: (0, i, 0)),
            row_block(BN, 16),
            full(D, D),
            full(1, D),
            full(D, D),
            full(1, D),
            full(D, D),
            full(R, D),
            full(1, D),
            full(D, D),
            full(D, D),
            full(D, D),
            full(1, D),
        ],
        out_specs=row_block(BN, D),
        out_shape=jax.ShapeDtypeStruct((N, D), jnp.float32),
    )(h, q, hs2, ab2, dg_,
      W_query, b_query.reshape(1, D), W_mem, b_mem.reshape(1, D),
      Wm1, Wm2, b_msg.reshape(1, D),
      Wa1, Wa2, Wa3, b_all.reshape(1, D))
    return out
